# baseline (speedup 1.0000x reference)
import functools

import jax
import jax.numpy as jnp
from jax import lax
from jax.experimental import pallas as pl
from jax.experimental.pallas import tpu as pltpu
from jax.experimental.pallas import tpu_sc as plsc

N_NODES = 10000
N_EDGES = 320000
D_NODE = 128
D_EDGE = 16
HID = 128
HALF = 64
NCORES = 2
NSUB = 16
EPT = N_EDGES // NSUB
EC = 40
NCHUNK = EPT // EC
EBLK = 3200
NEBLK = N_EDGES // EBLK
F32 = jnp.float32


def _mm_bias_body(x_ref, w_ref, b_ref, o_ref):
    o_ref[...] = (
        jnp.dot(x_ref[...], w_ref[...], preferred_element_type=F32, precision=jax.lax.Precision.HIGHEST)
        + b_ref[...]
    )


def _embed_h(x, w, b):
    return pl.pallas_call(
        _mm_bias_body,
        out_shape=jax.ShapeDtypeStruct((N_NODES, HID), F32),
    )(x, w, b.reshape(1, HID))


def _embed_e(e, w, b):
    return pl.pallas_call(
        _mm_bias_body,
        grid=(NEBLK,),
        in_specs=[
            pl.BlockSpec((EBLK, D_EDGE), lambda i: (i, 0)),
            pl.BlockSpec((D_EDGE, HID), lambda i: (0, 0)),
            pl.BlockSpec((1, HID), lambda i: (0, 0)),
        ],
        out_specs=pl.BlockSpec((EBLK, HID), lambda i: (i, 0)),
        out_shape=jax.ShapeDtypeStruct((N_EDGES, HID), F32),
    )(e, w, b.reshape(1, HID))


def _node_pre_body(h_ref, wa_ref, wb_ref, wd_ref, we_ref, bias_ref,
                   ah_ref, db_ref, eh_ref):
    h = h_ref[...]
    ah_ref[...] = jnp.dot(h, wa_ref[...], preferred_element_type=F32, precision=jax.lax.Precision.HIGHEST) + bias_ref[0:1]
    eh_ref[...] = jnp.dot(h, we_ref[...], preferred_element_type=F32, precision=jax.lax.Precision.HIGHEST) + bias_ref[3:4]
    bh = jnp.dot(h, wb_ref[...], preferred_element_type=F32, precision=jax.lax.Precision.HIGHEST) + bias_ref[1:2]
    dh = jnp.dot(h, wd_ref[...], preferred_element_type=F32, precision=jax.lax.Precision.HIGHEST) + bias_ref[2:3]
    db_ref[...] = jnp.concatenate(
        [dh[:, :HALF], bh[:, :HALF], dh[:, HALF:], bh[:, HALF:]], axis=1)


def _node_pre(h, lp):
    bias = jnp.stack([lp['bA'], lp['bB'], lp['bD'], lp['bE']])
    return pl.pallas_call(
        _node_pre_body,
        out_shape=(
            jax.ShapeDtypeStruct((N_NODES, HID), F32),
            jax.ShapeDtypeStruct((N_NODES, 2 * HID), F32),
            jax.ShapeDtypeStruct((N_NODES, HID), F32),
        ),
    )(h, lp['A'], lp['B'], lp['D'], lp['E'], bias)


def _edge_mm_body(ef_ref, c_ref, b_ref, o_ref):
    r = jnp.dot(ef_ref[...], c_ref[...], preferred_element_type=F32, precision=jax.lax.Precision.HIGHEST) + b_ref[...]
    o_ref[0] = r[:, :HALF]
    o_ref[1] = r[:, HALF:]


def _edge_mm(ef, c, b):
    return pl.pallas_call(
        _edge_mm_body,
        grid=(NEBLK,),
        in_specs=[
            pl.BlockSpec((EBLK, HID), lambda i: (i, 0)),
            pl.BlockSpec((HID, HID), lambda i: (0, 0)),
            pl.BlockSpec((1, HID), lambda i: (0, 0)),
        ],
        out_specs=pl.BlockSpec((NCORES, EBLK, HALF), lambda i: (0, i, 0)),
        out_shape=jax.ShapeDtypeStruct((NCORES, N_EDGES, HALF), F32),
    )(ef, c, b.reshape(1, HID))


def _edge_post_body(epre_ref, ef_ref, g_ref, b_ref, o_ref, acc_ref):
    p = pl.program_id(0)
    i = pl.program_id(1)
    ep = jnp.concatenate([epre_ref[0], epre_ref[1]], axis=1)

    @pl.when(jnp.logical_and(p == 0, i == 0))
    def _():
        acc_ref[...] = jnp.zeros_like(acc_ref)

    @pl.when(p == 0)
    def _():
        acc_ref[0:1] += jnp.sum(ep, axis=0, keepdims=True)
        acc_ref[1:2] += jnp.sum(ep * ep, axis=0, keepdims=True)

    @pl.when(p == 1)
    def _():
        mu = acc_ref[0:1] / N_EDGES
        var = acc_ref[1:2] / N_EDGES - mu * mu
        e_new = g_ref[...] * (ep - mu) / jnp.sqrt(var + 1e-5) + b_ref[...]
        o_ref[...] = ef_ref[...] + jnp.maximum(e_new, 0.0)


def _edge_post(epre2, ef, gamma, beta):
    return pl.pallas_call(
        _edge_post_body,
        grid=(2, NEBLK),
        in_specs=[
            pl.BlockSpec((NCORES, EBLK, HALF), lambda p, i: (0, i, 0)),
            pl.BlockSpec((EBLK, HID), lambda p, i: (i, 0)),
            pl.BlockSpec((1, HID), lambda p, i: (0, 0)),
            pl.BlockSpec((1, HID), lambda p, i: (0, 0)),
        ],
        out_specs=pl.BlockSpec((EBLK, HID), lambda p, i: (i, 0)),
        out_shape=jax.ShapeDtypeStruct((N_EDGES, HID), F32),
        scratch_shapes=[pltpu.VMEM((2, HID), F32)],
    )(epre2, ef, gamma.reshape(1, HID), beta.reshape(1, HID))


def _node_post_body(h_ref, ah_ref, nd_ref, g_ref, b_ref, o_ref):
    num = jnp.concatenate([nd_ref[0][:, :HALF], nd_ref[1][:, :HALF]], axis=1)
    den = jnp.concatenate([nd_ref[0][:, HALF:], nd_ref[1][:, HALF:]], axis=1)
    hn = ah_ref[...] + num / (den + 1e-6)
    mu = jnp.mean(hn, axis=0, keepdims=True)
    var = jnp.mean((hn - mu) * (hn - mu), axis=0, keepdims=True)
    hb = g_ref[...] * (hn - mu) / jnp.sqrt(var + 1e-5) + b_ref[...]
    o_ref[...] = h_ref[...] + jnp.maximum(hb, 0.0)


def _node_post(h, ah, nd2, gamma, beta):
    return pl.pallas_call(
        _node_post_body,
        out_shape=jax.ShapeDtypeStruct((N_NODES, HID), F32),
    )(h, ah, nd2, gamma.reshape(1, HID), beta.reshape(1, HID))


def _head_body(h_ref, a_ref, w1_ref, b1_ref, w2_ref, b2_ref, o_ref):
    hg = jnp.mean(h_ref[...], axis=0, keepdims=True)
    z = jnp.concatenate([hg, a_ref[...]], axis=1)
    z1 = jnp.maximum(
        jnp.dot(z, w1_ref[...], preferred_element_type=F32, precision=jax.lax.Precision.HIGHEST) + b1_ref[...], 0.0)
    o_ref[...] = jnp.dot(z1, w2_ref[...], preferred_element_type=F32, precision=jax.lax.Precision.HIGHEST) + b2_ref[...]


def _head(h, a, p):
    return pl.pallas_call(
        _head_body,
        out_shape=jax.ShapeDtypeStruct((1, 1), F32),
    )(h, a, p['W1'], p['b1'].reshape(1, -1), p['W2'], p['b2'].reshape(1, 1))


def _sc_edge_body(db_ref, eh_ref, ce_ref, src_ref, dst_ref, z_ref,
                  epre_ref, nd_ref,
                  si0, si1, di0, di1, gs0, gs1, dsc0, dsc1,
                  dbb0, dbb1, ebb0, ebb1, ceb0, ceb1, msb0, msb1,
                  sem_i, sg0, sg1, so0, so1,
                  nd_acc):
    c = lax.axis_index("c")
    s = lax.axis_index("s")
    SI = (si0, si1)
    DI = (di0, di1)
    GS = (gs0, gs1)
    DSC = (dsc0, dsc1)
    DBB = (dbb0, dbb1)
    EBB = (ebb0, ebb1)
    CEB = (ceb0, ceb1)
    MSB = (msb0, msb1)
    SG = (sg0, sg1)
    SO = (so0, so1)

    @pl.when(s < 10)
    def _():
        r0 = pl.multiple_of(s * 1000, 8)
        pltpu.sync_copy(z_ref.at[pl.ds(r0, 1000)],
                        nd_acc.at[pl.ds(r0, 1000)])
    plsc.subcore_barrier()

    ch = pl.multiple_of(c * HALF, 16)
    tbase = s * EPT

    def issue_idx(k, b):
        base = pl.multiple_of(tbase + k * EC, 8)
        pltpu.async_copy(src_ref.at[pl.ds(base, EC)], SI[b], sem_i)
        pltpu.async_copy(dst_ref.at[pl.ds(base, EC)], DI[b], sem_i)

    def wait_idx(b):
        pltpu.make_async_copy(src_ref.at[pl.ds(0, EC)], SI[b], sem_i).wait()
        pltpu.make_async_copy(dst_ref.at[pl.ds(0, EC)], DI[b], sem_i).wait()

    def transform(b):
        for st in (0, 16, 24):
            sl = pl.ds(pl.multiple_of(st, 8), 16)
            GS[b][sl] = SI[b][sl] * 2 + c
            DSC[b][sl] = DI[b][sl]

    def issue_gathers(k, b):
        base = pl.multiple_of(tbase + k * EC, 8)
        pltpu.async_copy(db_ref.at[GS[b]], DBB[b], SG[b])
        pltpu.async_copy(eh_ref.at[DI[b]], EBB[b], SG[b])
        pltpu.async_copy(ce_ref.at[c, pl.ds(base, EC)], CEB[b], SG[b])

    def wait_gathers(b):
        pltpu.make_async_copy(db_ref.at[GS[b]], DBB[b], SG[b]).wait()
        pltpu.make_async_copy(eh_ref.at[DI[b]], EBB[b], SG[b]).wait()
        pltpu.make_async_copy(ce_ref.at[c, pl.ds(0, EC)], CEB[b], SG[b]).wait()

    def issue_outs(k, b):
        base = pl.multiple_of(tbase + k * EC, 8)
        pltpu.async_copy(CEB[b], epre_ref.at[c, pl.ds(base, EC)], SO[b])
        pltpu.sync_copy(MSB[b], nd_acc.at[DSC[b]], add=True)

    def wait_outs(b):
        pltpu.make_async_copy(ce_ref.at[c, pl.ds(0, EC)], CEB[b], SO[b]).wait()

    def compute(b):
        dbb, ebb, ceb, msb = DBB[b], EBB[b], CEB[b], MSB[b]

        @plsc.parallel_loop(0, EC, step=1, unroll=8)
        def row(r):
            for j in range(HALF // 16):
                sl = pl.ds(pl.multiple_of(j * 16, 8), 16)
                slh = pl.ds(pl.multiple_of(ch + j * 16, 16), 16)
                slb = pl.ds(pl.multiple_of(HALF + j * 16, 8), 16)
                ep = dbb[r, sl] + ebb[r, slh] + ceb[r, sl]
                ceb[r, sl] = ep
                sg = 1.0 / (1.0 + jnp.exp(-ep))
                msb[r, sl] = sg * dbb[r, slb]
                msb[r, slb] = sg

    def steady(kk, b, first):
        b1 = 1 - b
        issue_idx(kk + 1, b1)
        wait_gathers(b)
        compute(b)
        issue_outs(kk, b)
        if not first:
            wait_outs(b1)
        wait_idx(b1)
        transform(b1)
        issue_gathers(kk + 1, b1)

    issue_idx(0, 0)
    wait_idx(0)
    transform(0)
    issue_gathers(0, 0)
    steady(0, 0, True)

    @pl.loop(1, NCHUNK - 1, step=2)
    def pair(k0):
        steady(k0, 1, False)
        steady(k0 + 1, 0, False)

    wait_gathers(1)
    compute(1)
    issue_outs(NCHUNK - 1, 1)
    wait_outs(0)
    wait_outs(1)
    plsc.subcore_barrier()

    @pl.when(s < 10)
    def _():
        q0 = pl.multiple_of(s * 1000, 8)
        pltpu.sync_copy(nd_acc.at[pl.ds(q0, 1000)],
                        nd_ref.at[c, pl.ds(q0, 1000)])


@functools.partial(
    pl.kernel,
    out_type=(
        jax.ShapeDtypeStruct((NCORES, N_EDGES, HALF), F32),
        jax.ShapeDtypeStruct((NCORES, N_NODES, HID), F32),
    ),
    mesh=plsc.VectorSubcoreMesh(core_axis_name="c", subcore_axis_name="s"),
    scratch_types=[
        pltpu.VMEM((EC,), jnp.int32),
        pltpu.VMEM((EC,), jnp.int32),
        pltpu.VMEM((EC,), jnp.int32),
        pltpu.VMEM((EC,), jnp.int32),
        pltpu.VMEM((EC,), jnp.int32),
        pltpu.VMEM((EC,), jnp.int32),
        pltpu.VMEM((EC,), jnp.int32),
        pltpu.VMEM((EC,), jnp.int32),
        pltpu.VMEM((EC, HID), F32),
        pltpu.VMEM((EC, HID), F32),
        pltpu.VMEM((EC, HID), F32),
        pltpu.VMEM((EC, HID), F32),
        pltpu.VMEM((EC, HALF), F32),
        pltpu.VMEM((EC, HALF), F32),
        pltpu.VMEM((EC, HID), F32),
        pltpu.VMEM((EC, HID), F32),
        pltpu.SemaphoreType.DMA,
        pltpu.SemaphoreType.DMA,
        pltpu.SemaphoreType.DMA,
        pltpu.SemaphoreType.DMA,
        pltpu.SemaphoreType.DMA,
        pltpu.VMEM_SHARED((N_NODES, HID), F32),
    ],
)
def _sc_edge(db, eh, ce2, src, dst, zeros,
             epre2, nd2, *scratch):
    _sc_edge_body(db, eh, ce2, src, dst, zeros,
                  epre2, nd2, *scratch)


def kernel(x, e, edge_index, a, params1, params2):
    src = edge_index[0].astype(jnp.int32)
    dst = edge_index[1].astype(jnp.int32)
    zeros = jnp.zeros((N_NODES, HID), F32)
    nets = []
    for p in (params1, params2):
        nets.append({
            'p': p,
            'h': _embed_h(x, p['W_h'], p['b_h']),
            'ef': _embed_e(e, p['W_e'], p['b_e']),
        })
    nl = len(params1['layers'])
    for li in range(nl):
        pre = []
        for n in nets:
            lp = n['p']['layers'][li]
            ah, db, eh = _node_pre(n['h'], lp)
            ce2 = _edge_mm(n['ef'], lp['C'], lp['bC'])
            pre.append((ah, db, eh, ce2))
        for n, (ah, db, eh, ce2) in zip(nets, pre):
            lp = n['p']['layers'][li]
            epre2, nd2 = _sc_edge(
                db.reshape(2 * N_NODES, HID), eh, ce2, src, dst, zeros)
            n['h'] = _node_post(n['h'], ah, nd2, lp['gamma_h'], lp['beta_h'])
            if li + 1 < nl:
                n['ef'] = _edge_post(epre2, n['ef'], lp['gamma_e'], lp['beta_e'])
    q1 = _head(nets[0]['h'], a, params1)
    q2 = _head(nets[1]['h'], a, params2)
    return (q1, q2)


# speedup vs baseline: 1.0105x; 1.0105x over previous
import functools

import jax
import jax.numpy as jnp
from jax import lax
from jax.experimental import pallas as pl
from jax.experimental.pallas import tpu as pltpu
from jax.experimental.pallas import tpu_sc as plsc

N_NODES = 10000
N_EDGES = 320000
D_NODE = 128
D_EDGE = 16
HID = 128
HALF = 64
NCORES = 2
NSUB = 16
EPT = N_EDGES // NSUB
EC = 40
NCHUNK = EPT // EC
EBLK = 3200
NEBLK = N_EDGES // EBLK
F32 = jnp.float32


def _mm_bias_body(x_ref, w_ref, b_ref, o_ref):
    o_ref[...] = (
        jnp.dot(x_ref[...], w_ref[...], preferred_element_type=F32)
        + b_ref[...]
    )


def _embed_h(x, w, b):
    return pl.pallas_call(
        _mm_bias_body,
        out_shape=jax.ShapeDtypeStruct((N_NODES, HID), F32),
    )(x, w, b.reshape(1, HID))


def _embed_e(e, w, b):
    return pl.pallas_call(
        _mm_bias_body,
        grid=(NEBLK,),
        in_specs=[
            pl.BlockSpec((EBLK, D_EDGE), lambda i: (i, 0)),
            pl.BlockSpec((D_EDGE, HID), lambda i: (0, 0)),
            pl.BlockSpec((1, HID), lambda i: (0, 0)),
        ],
        out_specs=pl.BlockSpec((EBLK, HID), lambda i: (i, 0)),
        out_shape=jax.ShapeDtypeStruct((N_EDGES, HID), F32),
    )(e, w, b.reshape(1, HID))


def _node_pre_body(h_ref, wa_ref, wb_ref, wd_ref, we_ref, bias_ref,
                   ah_ref, db_ref, eh_ref):
    h = h_ref[...]
    ah_ref[...] = jnp.dot(h, wa_ref[...], preferred_element_type=F32) + bias_ref[0:1]
    eh_ref[...] = jnp.dot(h, we_ref[...], preferred_element_type=F32) + bias_ref[3:4]
    bh = jnp.dot(h, wb_ref[...], preferred_element_type=F32) + bias_ref[1:2]
    dh = jnp.dot(h, wd_ref[...], preferred_element_type=F32) + bias_ref[2:3]
    db_ref[...] = jnp.concatenate(
        [dh[:, :HALF], bh[:, :HALF], dh[:, HALF:], bh[:, HALF:]], axis=1)


def _node_pre(h, lp):
    bias = jnp.stack([lp['bA'], lp['bB'], lp['bD'], lp['bE']])
    return pl.pallas_call(
        _node_pre_body,
        out_shape=(
            jax.ShapeDtypeStruct((N_NODES, HID), F32),
            jax.ShapeDtypeStruct((N_NODES, 2 * HID), F32),
            jax.ShapeDtypeStruct((N_NODES, HID), F32),
        ),
    )(h, lp['A'], lp['B'], lp['D'], lp['E'], bias)


def _edge_mm_body(ef_ref, c_ref, b_ref, o_ref):
    r = jnp.dot(ef_ref[...], c_ref[...], preferred_element_type=F32) + b_ref[...]
    o_ref[0] = r[:, :HALF]
    o_ref[1] = r[:, HALF:]


def _edge_mm(ef, c, b):
    return pl.pallas_call(
        _edge_mm_body,
        grid=(NEBLK,),
        in_specs=[
            pl.BlockSpec((EBLK, HID), lambda i: (i, 0)),
            pl.BlockSpec((HID, HID), lambda i: (0, 0)),
            pl.BlockSpec((1, HID), lambda i: (0, 0)),
        ],
        out_specs=pl.BlockSpec((NCORES, EBLK, HALF), lambda i: (0, i, 0)),
        out_shape=jax.ShapeDtypeStruct((NCORES, N_EDGES, HALF), F32),
    )(ef, c, b.reshape(1, HID))


def _edge_post_body(epre_ref, ef_ref, g_ref, b_ref, o_ref, acc_ref):
    p = pl.program_id(0)
    i = pl.program_id(1)
    ep = jnp.concatenate([epre_ref[0], epre_ref[1]], axis=1)

    @pl.when(jnp.logical_and(p == 0, i == 0))
    def _():
        acc_ref[...] = jnp.zeros_like(acc_ref)

    @pl.when(p == 0)
    def _():
        acc_ref[0:1] += jnp.sum(ep, axis=0, keepdims=True)
        acc_ref[1:2] += jnp.sum(ep * ep, axis=0, keepdims=True)

    @pl.when(p == 1)
    def _():
        mu = acc_ref[0:1] / N_EDGES
        var = acc_ref[1:2] / N_EDGES - mu * mu
        e_new = g_ref[...] * (ep - mu) / jnp.sqrt(var + 1e-5) + b_ref[...]
        o_ref[...] = ef_ref[...] + jnp.maximum(e_new, 0.0)


def _edge_post(epre2, ef, gamma, beta):
    return pl.pallas_call(
        _edge_post_body,
        grid=(2, NEBLK),
        in_specs=[
            pl.BlockSpec((NCORES, EBLK, HALF), lambda p, i: (0, i, 0)),
            pl.BlockSpec((EBLK, HID), lambda p, i: (i, 0)),
            pl.BlockSpec((1, HID), lambda p, i: (0, 0)),
            pl.BlockSpec((1, HID), lambda p, i: (0, 0)),
        ],
        out_specs=pl.BlockSpec((EBLK, HID), lambda p, i: (i, 0)),
        out_shape=jax.ShapeDtypeStruct((N_EDGES, HID), F32),
        scratch_shapes=[pltpu.VMEM((2, HID), F32)],
    )(epre2, ef, gamma.reshape(1, HID), beta.reshape(1, HID))


def _node_post_body(h_ref, ah_ref, nd_ref, g_ref, b_ref, o_ref):
    num = jnp.concatenate([nd_ref[0][:, :HALF], nd_ref[1][:, :HALF]], axis=1)
    den = jnp.concatenate([nd_ref[0][:, HALF:], nd_ref[1][:, HALF:]], axis=1)
    hn = ah_ref[...] + num / (den + 1e-6)
    mu = jnp.mean(hn, axis=0, keepdims=True)
    var = jnp.mean((hn - mu) * (hn - mu), axis=0, keepdims=True)
    hb = g_ref[...] * (hn - mu) / jnp.sqrt(var + 1e-5) + b_ref[...]
    o_ref[...] = h_ref[...] + jnp.maximum(hb, 0.0)


def _node_post(h, ah, nd2, gamma, beta):
    return pl.pallas_call(
        _node_post_body,
        out_shape=jax.ShapeDtypeStruct((N_NODES, HID), F32),
    )(h, ah, nd2, gamma.reshape(1, HID), beta.reshape(1, HID))


def _head_body(h_ref, a_ref, w1_ref, b1_ref, w2_ref, b2_ref, o_ref):
    hg = jnp.mean(h_ref[...], axis=0, keepdims=True)
    z = jnp.concatenate([hg, a_ref[...]], axis=1)
    z1 = jnp.maximum(
        jnp.dot(z, w1_ref[...], preferred_element_type=F32) + b1_ref[...], 0.0)
    o_ref[...] = jnp.dot(z1, w2_ref[...], preferred_element_type=F32) + b2_ref[...]


def _head(h, a, p):
    return pl.pallas_call(
        _head_body,
        out_shape=jax.ShapeDtypeStruct((1, 1), F32),
    )(h, a, p['W1'], p['b1'].reshape(1, -1), p['W2'], p['b2'].reshape(1, 1))


def _sc_edge_body(db_ref, eh_ref, ce_ref, src_ref, dst_ref, z_ref,
                  epre_ref, nd_ref,
                  si0, si1, di0, di1, gs0, gs1, dsc0, dsc1,
                  dbb0, dbb1, ebb0, ebb1, ceb0, ceb1, msb0, msb1,
                  sem_i, sg0, sg1, so0, so1,
                  nd_acc):
    c = lax.axis_index("c")
    s = lax.axis_index("s")
    SI = (si0, si1)
    DI = (di0, di1)
    GS = (gs0, gs1)
    DSC = (dsc0, dsc1)
    DBB = (dbb0, dbb1)
    EBB = (ebb0, ebb1)
    CEB = (ceb0, ceb1)
    MSB = (msb0, msb1)
    SG = (sg0, sg1)
    SO = (so0, so1)

    @pl.when(s < 10)
    def _():
        r0 = pl.multiple_of(s * 1000, 8)
        pltpu.sync_copy(z_ref.at[pl.ds(r0, 1000)],
                        nd_acc.at[pl.ds(r0, 1000)])
    plsc.subcore_barrier()

    ch = pl.multiple_of(c * HALF, 16)
    tbase = s * EPT

    def issue_idx(k, b):
        base = pl.multiple_of(tbase + k * EC, 8)
        pltpu.async_copy(src_ref.at[pl.ds(base, EC)], SI[b], sem_i)
        pltpu.async_copy(dst_ref.at[pl.ds(base, EC)], DI[b], sem_i)

    def wait_idx(b):
        pltpu.make_async_copy(src_ref.at[pl.ds(0, EC)], SI[b], sem_i).wait()
        pltpu.make_async_copy(dst_ref.at[pl.ds(0, EC)], DI[b], sem_i).wait()

    def transform(b):
        for st in (0, 16, 24):
            sl = pl.ds(pl.multiple_of(st, 8), 16)
            GS[b][sl] = SI[b][sl] * 2 + c
            DSC[b][sl] = DI[b][sl]

    def issue_gathers(k, b):
        base = pl.multiple_of(tbase + k * EC, 8)
        pltpu.async_copy(db_ref.at[GS[b]], DBB[b], SG[b])
        pltpu.async_copy(eh_ref.at[DI[b]], EBB[b], SG[b])
        pltpu.async_copy(ce_ref.at[c, pl.ds(base, EC)], CEB[b], SG[b])

    def wait_gathers(b):
        pltpu.make_async_copy(db_ref.at[GS[b]], DBB[b], SG[b]).wait()
        pltpu.make_async_copy(eh_ref.at[DI[b]], EBB[b], SG[b]).wait()
        pltpu.make_async_copy(ce_ref.at[c, pl.ds(0, EC)], CEB[b], SG[b]).wait()

    def issue_outs(k, b):
        base = pl.multiple_of(tbase + k * EC, 8)
        pltpu.async_copy(CEB[b], epre_ref.at[c, pl.ds(base, EC)], SO[b])
        pltpu.sync_copy(MSB[b], nd_acc.at[DSC[b]], add=True)

    def wait_outs(b):
        pltpu.make_async_copy(ce_ref.at[c, pl.ds(0, EC)], CEB[b], SO[b]).wait()

    def compute(b):
        dbb, ebb, ceb, msb = DBB[b], EBB[b], CEB[b], MSB[b]

        @plsc.parallel_loop(0, EC, step=1, unroll=8)
        def row(r):
            for j in range(HALF // 16):
                sl = pl.ds(pl.multiple_of(j * 16, 8), 16)
                slh = pl.ds(pl.multiple_of(ch + j * 16, 16), 16)
                slb = pl.ds(pl.multiple_of(HALF + j * 16, 8), 16)
                ep = dbb[r, sl] + ebb[r, slh] + ceb[r, sl]
                ceb[r, sl] = ep
                sg = 1.0 / (1.0 + jnp.exp(-ep))
                msb[r, sl] = sg * dbb[r, slb]
                msb[r, slb] = sg

    def steady(kk, b, first):
        b1 = 1 - b
        issue_idx(kk + 1, b1)
        wait_gathers(b)
        compute(b)
        issue_outs(kk, b)
        if not first:
            wait_outs(b1)
        wait_idx(b1)
        transform(b1)
        issue_gathers(kk + 1, b1)

    issue_idx(0, 0)
    wait_idx(0)
    transform(0)
    issue_gathers(0, 0)
    steady(0, 0, True)

    @pl.loop(1, NCHUNK - 1, step=2)
    def pair(k0):
        steady(k0, 1, False)
        steady(k0 + 1, 0, False)

    wait_gathers(1)
    compute(1)
    issue_outs(NCHUNK - 1, 1)
    wait_outs(0)
    wait_outs(1)
    plsc.subcore_barrier()

    @pl.when(s < 10)
    def _():
        q0 = pl.multiple_of(s * 1000, 8)
        pltpu.sync_copy(nd_acc.at[pl.ds(q0, 1000)],
                        nd_ref.at[c, pl.ds(q0, 1000)])


@functools.partial(
    pl.kernel,
    out_type=(
        jax.ShapeDtypeStruct((NCORES, N_EDGES, HALF), F32),
        jax.ShapeDtypeStruct((NCORES, N_NODES, HID), F32),
    ),
    mesh=plsc.VectorSubcoreMesh(core_axis_name="c", subcore_axis_name="s"),
    scratch_types=[
        pltpu.VMEM((EC,), jnp.int32),
        pltpu.VMEM((EC,), jnp.int32),
        pltpu.VMEM((EC,), jnp.int32),
        pltpu.VMEM((EC,), jnp.int32),
        pltpu.VMEM((EC,), jnp.int32),
        pltpu.VMEM((EC,), jnp.int32),
        pltpu.VMEM((EC,), jnp.int32),
        pltpu.VMEM((EC,), jnp.int32),
        pltpu.VMEM((EC, HID), F32),
        pltpu.VMEM((EC, HID), F32),
        pltpu.VMEM((EC, HID), F32),
        pltpu.VMEM((EC, HID), F32),
        pltpu.VMEM((EC, HALF), F32),
        pltpu.VMEM((EC, HALF), F32),
        pltpu.VMEM((EC, HID), F32),
        pltpu.VMEM((EC, HID), F32),
        pltpu.SemaphoreType.DMA,
        pltpu.SemaphoreType.DMA,
        pltpu.SemaphoreType.DMA,
        pltpu.SemaphoreType.DMA,
        pltpu.SemaphoreType.DMA,
        pltpu.VMEM_SHARED((N_NODES, HID), F32),
    ],
)
def _sc_edge(db, eh, ce2, src, dst, zeros,
             epre2, nd2, *scratch):
    _sc_edge_body(db, eh, ce2, src, dst, zeros,
                  epre2, nd2, *scratch)


def _gated_gcn(x, e, src, dst, zeros, p):
    h = _embed_h(x, p['W_h'], p['b_h'])
    ef = _embed_e(e, p['W_e'], p['b_e'])
    nl = len(p['layers'])
    for li, lp in enumerate(p['layers']):
        ah, db, eh = _node_pre(h, lp)
        ce2 = _edge_mm(ef, lp['C'], lp['bC'])
        epre2, nd2 = _sc_edge(
            db.reshape(2 * N_NODES, HID), eh, ce2, src, dst, zeros)
        h = _node_post(h, ah, nd2, lp['gamma_h'], lp['beta_h'])
        if li + 1 < nl:
            ef = _edge_post(epre2, ef, lp['gamma_e'], lp['beta_e'])
    return h


def kernel(x, e, edge_index, a, params1, params2):
    src = edge_index[0].astype(jnp.int32)
    dst = edge_index[1].astype(jnp.int32)
    zeros = jnp.zeros((N_NODES, HID), F32)
    h1 = _gated_gcn(x, e, src, dst, zeros, params1)
    q1 = _head(h1, a, params1)
    h2 = _gated_gcn(x, e, src, dst, zeros, params2)
    q2 = _head(h2, a, params2)
    return (q1, q2)


# speedup vs baseline: 1.0267x; 1.0160x over previous
import functools

import jax
import jax.numpy as jnp
from jax import lax
from jax.experimental import pallas as pl
from jax.experimental.pallas import tpu as pltpu
from jax.experimental.pallas import tpu_sc as plsc

N_NODES = 10000
N_EDGES = 320000
D_NODE = 128
D_EDGE = 16
HID = 128
HALF = 64
NCORES = 2
NSUB = 16
EPT = N_EDGES // NSUB
EC = 40
NCHUNK = EPT // EC
EBLK = 3200
NEBLK = N_EDGES // EBLK
F32 = jnp.float32


def _mm_bias_body(x_ref, w_ref, b_ref, o_ref):
    o_ref[...] = (
        jnp.dot(x_ref[...], w_ref[...], preferred_element_type=F32)
        + b_ref[...]
    )


def _embed_h(x, w, b):
    return pl.pallas_call(
        _mm_bias_body,
        out_shape=jax.ShapeDtypeStruct((N_NODES, HID), F32),
    )(x, w, b.reshape(1, HID))


def _embed_e(e, w, b):
    return pl.pallas_call(
        _mm_bias_body,
        grid=(NEBLK,),
        in_specs=[
            pl.BlockSpec((EBLK, D_EDGE), lambda i: (i, 0)),
            pl.BlockSpec((D_EDGE, HID), lambda i: (0, 0)),
            pl.BlockSpec((1, HID), lambda i: (0, 0)),
        ],
        out_specs=pl.BlockSpec((EBLK, HID), lambda i: (i, 0)),
        out_shape=jax.ShapeDtypeStruct((N_EDGES, HID), F32),
    )(e, w, b.reshape(1, HID))


def _node_pre_body(h_ref, wa_ref, wb_ref, wd_ref, we_ref, bias_ref,
                   ah_ref, db_ref, eh_ref):
    h = h_ref[...]
    ah_ref[...] = jnp.dot(h, wa_ref[...], preferred_element_type=F32) + bias_ref[0:1]
    eh_ref[...] = jnp.dot(h, we_ref[...], preferred_element_type=F32) + bias_ref[3:4]
    bh = jnp.dot(h, wb_ref[...], preferred_element_type=F32) + bias_ref[1:2]
    dh = jnp.dot(h, wd_ref[...], preferred_element_type=F32) + bias_ref[2:3]
    db_ref[...] = jnp.concatenate(
        [dh[:, :HALF], bh[:, :HALF], dh[:, HALF:], bh[:, HALF:]], axis=1)


def _node_pre(h, lp):
    bias = jnp.stack([lp['bA'], lp['bB'], lp['bD'], lp['bE']])
    return pl.pallas_call(
        _node_pre_body,
        out_shape=(
            jax.ShapeDtypeStruct((N_NODES, HID), F32),
            jax.ShapeDtypeStruct((N_NODES, 2 * HID), F32),
            jax.ShapeDtypeStruct((N_NODES, HID), F32),
        ),
    )(h, lp['A'], lp['B'], lp['D'], lp['E'], bias)


def _edge_mm_body(ef_ref, c_ref, b_ref, o_ref):
    r = jnp.dot(ef_ref[...], c_ref[...], preferred_element_type=F32) + b_ref[...]
    o_ref[0] = r[:, :HALF]
    o_ref[1] = r[:, HALF:]


def _edge_mm(ef, c, b):
    return pl.pallas_call(
        _edge_mm_body,
        grid=(NEBLK,),
        in_specs=[
            pl.BlockSpec((EBLK, HID), lambda i: (i, 0)),
            pl.BlockSpec((HID, HID), lambda i: (0, 0)),
            pl.BlockSpec((1, HID), lambda i: (0, 0)),
        ],
        out_specs=pl.BlockSpec((NCORES, EBLK, HALF), lambda i: (0, i, 0)),
        out_shape=jax.ShapeDtypeStruct((NCORES, N_EDGES, HALF), F32),
    )(ef, c, b.reshape(1, HID))


def _edge_post_body(epre_ref, ef_ref, g_ref, b_ref, o_ref, acc_ref):
    p = pl.program_id(0)
    i = pl.program_id(1)
    ep = jnp.concatenate([epre_ref[0], epre_ref[1]], axis=1)

    @pl.when(jnp.logical_and(p == 0, i == 0))
    def _():
        acc_ref[...] = jnp.zeros_like(acc_ref)

    @pl.when(p == 0)
    def _():
        acc_ref[0:1] += jnp.sum(ep, axis=0, keepdims=True)
        acc_ref[1:2] += jnp.sum(ep * ep, axis=0, keepdims=True)

    @pl.when(p == 1)
    def _():
        mu = acc_ref[0:1] / N_EDGES
        var = acc_ref[1:2] / N_EDGES - mu * mu
        e_new = g_ref[...] * (ep - mu) / jnp.sqrt(var + 1e-5) + b_ref[...]
        o_ref[...] = ef_ref[...] + jnp.maximum(e_new, 0.0)


def _edge_post(epre2, ef, gamma, beta):
    return pl.pallas_call(
        _edge_post_body,
        grid=(2, NEBLK),
        in_specs=[
            pl.BlockSpec((NCORES, EBLK, HALF), lambda p, i: (0, i, 0)),
            pl.BlockSpec((EBLK, HID), lambda p, i: (i, 0)),
            pl.BlockSpec((1, HID), lambda p, i: (0, 0)),
            pl.BlockSpec((1, HID), lambda p, i: (0, 0)),
        ],
        out_specs=pl.BlockSpec((EBLK, HID), lambda p, i: (i, 0)),
        out_shape=jax.ShapeDtypeStruct((N_EDGES, HID), F32),
        scratch_shapes=[pltpu.VMEM((2, HID), F32)],
    )(epre2, ef, gamma.reshape(1, HID), beta.reshape(1, HID))


def _node_post_body(h_ref, ah_ref, nd_ref, g_ref, b_ref, o_ref):
    num = jnp.concatenate([nd_ref[0][:, :HALF], nd_ref[1][:, :HALF]], axis=1)
    den = jnp.concatenate([nd_ref[0][:, HALF:], nd_ref[1][:, HALF:]], axis=1)
    hn = ah_ref[...] + num / (den + 1e-6)
    mu = jnp.mean(hn, axis=0, keepdims=True)
    var = jnp.mean((hn - mu) * (hn - mu), axis=0, keepdims=True)
    hb = g_ref[...] * (hn - mu) / jnp.sqrt(var + 1e-5) + b_ref[...]
    o_ref[...] = h_ref[...] + jnp.maximum(hb, 0.0)


def _node_post(h, ah, nd2, gamma, beta):
    return pl.pallas_call(
        _node_post_body,
        out_shape=jax.ShapeDtypeStruct((N_NODES, HID), F32),
    )(h, ah, nd2, gamma.reshape(1, HID), beta.reshape(1, HID))


def _head_body(h_ref, a_ref, w1_ref, b1_ref, w2_ref, b2_ref, o_ref):
    hg = jnp.mean(h_ref[...], axis=0, keepdims=True)
    z = jnp.concatenate([hg, a_ref[...]], axis=1)
    z1 = jnp.maximum(
        jnp.dot(z, w1_ref[...], preferred_element_type=F32) + b1_ref[...], 0.0)
    o_ref[...] = jnp.dot(z1, w2_ref[...], preferred_element_type=F32) + b2_ref[...]


def _head(h, a, p):
    return pl.pallas_call(
        _head_body,
        out_shape=jax.ShapeDtypeStruct((1, 1), F32),
    )(h, a, p['W1'], p['b1'].reshape(1, -1), p['W2'], p['b2'].reshape(1, 1))


def _sc_edge_body(db_ref, eh_ref, ce_ref, src_ref, dst_ref, z_ref,
                  epre_ref, nd_ref,
                  si0, si1, di0, di1, gs0, gs1, dscp,
                  dbb0, dbb1, ebb0, ebb1, ceb0, ceb1, msp,
                  sem_i, sg0, sg1, so0, so1,
                  nd_acc):
    c = lax.axis_index("c")
    s = lax.axis_index("s")
    SI = (si0, si1)
    DI = (di0, di1)
    GS = (gs0, gs1)
    DBB = (dbb0, dbb1)
    EBB = (ebb0, ebb1)
    CEB = (ceb0, ceb1)
    SG = (sg0, sg1)
    SO = (so0, so1)

    @pl.when(s < 10)
    def _():
        r0 = pl.multiple_of(s * 1000, 8)
        pltpu.sync_copy(z_ref.at[pl.ds(r0, 1000)],
                        nd_acc.at[pl.ds(r0, 1000)])
    plsc.subcore_barrier()

    ch = pl.multiple_of(c * HALF, 16)
    tbase = s * EPT

    def issue_idx(k, b):
        base = pl.multiple_of(tbase + k * EC, 8)
        pltpu.async_copy(src_ref.at[pl.ds(base, EC)], SI[b], sem_i)
        pltpu.async_copy(dst_ref.at[pl.ds(base, EC)], DI[b], sem_i)

    def wait_idx(b):
        pltpu.make_async_copy(src_ref.at[pl.ds(0, EC)], SI[b], sem_i).wait()
        pltpu.make_async_copy(dst_ref.at[pl.ds(0, EC)], DI[b], sem_i).wait()

    def transform(b):
        for st in (0, 16, 24):
            sl = pl.ds(pl.multiple_of(st, 8), 16)
            slp = pl.ds(pl.multiple_of(EC * b + st, 8), 16)
            GS[b][sl] = SI[b][sl] * 2 + c
            dscp[slp] = DI[b][sl]

    def issue_gathers(k, b):
        base = pl.multiple_of(tbase + k * EC, 8)
        pltpu.async_copy(db_ref.at[GS[b]], DBB[b], SG[b])
        pltpu.async_copy(eh_ref.at[DI[b]], EBB[b], SG[b])
        pltpu.async_copy(ce_ref.at[c, pl.ds(base, EC)], CEB[b], SG[b])

    def wait_gathers(b):
        pltpu.make_async_copy(db_ref.at[GS[b]], DBB[b], SG[b]).wait()
        pltpu.make_async_copy(eh_ref.at[DI[b]], EBB[b], SG[b]).wait()
        pltpu.make_async_copy(ce_ref.at[c, pl.ds(0, EC)], CEB[b], SG[b]).wait()

    def wait_outs(b):
        pltpu.make_async_copy(ce_ref.at[c, pl.ds(0, EC)], CEB[b], SO[b]).wait()

    def compute(b):
        dbb, ebb, ceb = DBB[b], EBB[b], CEB[b]
        off = EC * b

        @plsc.parallel_loop(0, EC, step=1, unroll=8)
        def row(r):
            for j in range(HALF // 16):
                sl = pl.ds(pl.multiple_of(j * 16, 8), 16)
                slh = pl.ds(pl.multiple_of(ch + j * 16, 16), 16)
                slb = pl.ds(pl.multiple_of(HALF + j * 16, 8), 16)
                ep = dbb[r, sl] + ebb[r, slh] + ceb[r, sl]
                ceb[r, sl] = ep
                sg = 1.0 / (1.0 + jnp.exp(-ep))
                msp[off + r, sl] = sg * dbb[r, slb]
                msp[off + r, slb] = sg

    def steady(kk, b):
        b1 = 1 - b
        knext = jnp.where(kk + 1 < NCHUNK, kk + 1, 0)
        issue_idx(knext, b1)
        wait_gathers(b)
        compute(b)
        base = pl.multiple_of(tbase + kk * EC, 8)
        pltpu.async_copy(CEB[b], epre_ref.at[c, pl.ds(base, EC)], SO[b])
        if b == 1:
            pltpu.sync_copy(msp, nd_acc.at[dscp], add=True)
        wait_outs(b1)
        wait_idx(b1)
        transform(b1)
        issue_gathers(knext, b1)

    issue_idx(0, 0)
    wait_idx(0)
    transform(0)
    issue_gathers(0, 0)
    pltpu.async_copy(CEB[1], epre_ref.at[c, pl.ds(tbase, EC)], SO[0])
    pltpu.async_copy(CEB[1], epre_ref.at[c, pl.ds(tbase + EC, EC)], SO[1])

    @pl.loop(0, NCHUNK, step=2)
    def pair(k0):
        steady(k0, 0)
        steady(k0 + 1, 1)

    wait_outs(0)
    wait_outs(1)
    wait_gathers(0)
    plsc.subcore_barrier()

    @pl.when(s < 10)
    def _():
        q0 = pl.multiple_of(s * 1000, 8)
        pltpu.sync_copy(nd_acc.at[pl.ds(q0, 1000)],
                        nd_ref.at[c, pl.ds(q0, 1000)])


@functools.partial(
    pl.kernel,
    out_type=(
        jax.ShapeDtypeStruct((NCORES, N_EDGES, HALF), F32),
        jax.ShapeDtypeStruct((NCORES, N_NODES, HID), F32),
    ),
    mesh=plsc.VectorSubcoreMesh(core_axis_name="c", subcore_axis_name="s"),
    scratch_types=[
        pltpu.VMEM((EC,), jnp.int32),
        pltpu.VMEM((EC,), jnp.int32),
        pltpu.VMEM((EC,), jnp.int32),
        pltpu.VMEM((EC,), jnp.int32),
        pltpu.VMEM((EC,), jnp.int32),
        pltpu.VMEM((EC,), jnp.int32),
        pltpu.VMEM((2 * EC,), jnp.int32),
        pltpu.VMEM((EC, HID), F32),
        pltpu.VMEM((EC, HID), F32),
        pltpu.VMEM((EC, HID), F32),
        pltpu.VMEM((EC, HID), F32),
        pltpu.VMEM((EC, HALF), F32),
        pltpu.VMEM((EC, HALF), F32),
        pltpu.VMEM((2 * EC, HID), F32),
        pltpu.SemaphoreType.DMA,
        pltpu.SemaphoreType.DMA,
        pltpu.SemaphoreType.DMA,
        pltpu.SemaphoreType.DMA,
        pltpu.SemaphoreType.DMA,
        pltpu.VMEM_SHARED((N_NODES, HID), F32),
    ],
)
def _sc_edge(db, eh, ce2, src, dst, zeros,
             epre2, nd2, *scratch):
    _sc_edge_body(db, eh, ce2, src, dst, zeros,
                  epre2, nd2, *scratch)


def _gated_gcn(x, e, src, dst, zeros, p):
    h = _embed_h(x, p['W_h'], p['b_h'])
    ef = _embed_e(e, p['W_e'], p['b_e'])
    nl = len(p['layers'])
    for li, lp in enumerate(p['layers']):
        ah, db, eh = _node_pre(h, lp)
        ce2 = _edge_mm(ef, lp['C'], lp['bC'])
        epre2, nd2 = _sc_edge(
            db.reshape(2 * N_NODES, HID), eh, ce2, src, dst, zeros)
        h = _node_post(h, ah, nd2, lp['gamma_h'], lp['beta_h'])
        if li + 1 < nl:
            ef = _edge_post(epre2, ef, lp['gamma_e'], lp['beta_e'])
    return h


def kernel(x, e, edge_index, a, params1, params2):
    src = edge_index[0].astype(jnp.int32)
    dst = edge_index[1].astype(jnp.int32)
    zeros = jnp.zeros((N_NODES, HID), F32)
    h1 = _gated_gcn(x, e, src, dst, zeros, params1)
    q1 = _head(h1, a, params1)
    h2 = _gated_gcn(x, e, src, dst, zeros, params2)
    q2 = _head(h2, a, params2)
    return (q1, q2)
